# baseline (speedup 1.0000x reference)
import jax
import jax.numpy as jnp
from jax.experimental import pallas as pl
from jax.experimental.pallas import tpu as pltpu


def _nodes_kernel(rf_ref, w_ref, b_ref, out_ref):
    rf = rf_ref[0]
    nodes = jax.lax.dot_general(
        rf, w_ref[...], (((1,), (1,)), ((), ())),
        preferred_element_type=jnp.float32)
    out_ref[0] = nodes + b_ref[...]


def _main_kernel(nodes_ref, wmsg_ref, bmsg_ref, wout_ref, bout_ref,
                 out_ref, vals_ref, *, R, K, N):
    rb = pl.program_id(1)
    nodes_all = nodes_ref[0]
    rows = nodes_ref[0, pl.ds(rb * R, R), :]

    sim = jax.lax.dot_general(
        rows, nodes_all, (((1,), (1,)), ((), ())),
        preferred_element_type=jnp.float32)

    col = jax.lax.broadcasted_iota(jnp.int32, (R, N), 1)
    row_g = rb * R + jax.lax.broadcasted_iota(jnp.int32, (R, N), 0)
    sim = sim + jnp.where(col == row_g, jnp.float32(-1e9), jnp.float32(0.0))

    lane_k = jax.lax.broadcasted_iota(jnp.int32, (R, K), 1)
    vals = jnp.zeros((R, K), jnp.float32)
    simw = sim
    C = 32
    L = N // C
    m = jnp.float32(0)
    for kk in range(K):
        cmax = jnp.max(simw.reshape(R, C, L), axis=2)
        m = jnp.max(cmax, axis=1, keepdims=True)
        vals = jnp.where(lane_k == kk, m, vals)
        if kk < K - 1:
            simw = jnp.where(simw == m, -jnp.inf, simw)
    sel = jnp.logical_or(simw == -jnp.inf, simw == m).astype(jnp.float32)

    neigh = jax.lax.dot_general(
        sel, nodes_all, (((1,), (0,)), ((), ())),
        preferred_element_type=jnp.float32) * jnp.float32(1.0 / K)

    msgs = jax.lax.dot_general(
        neigh, wmsg_ref[...], (((1,), (1,)), ((), ())),
        preferred_element_type=jnp.float32) + bmsg_ref[...]
    msgs = jnp.maximum(msgs, 0.0)
    updated = rows + msgs

    out = jax.lax.dot_general(
        updated, wout_ref[...], (((1,), (1,)), ((), ())),
        preferred_element_type=jnp.float32) + bout_ref[...]
    out_ref[0] = out
    vals_ref[0] = vals


def kernel(region_features, W_node, b_node, W_msg, b_msg, W_out, b_out):
    B, N, D = region_features.shape
    H = W_node.shape[0]
    K = min(6, N - 1)
    R = 256

    nodes = pl.pallas_call(
        _nodes_kernel,
        grid=(B,),
        in_specs=[
            pl.BlockSpec((1, N, D), lambda b: (b, 0, 0)),
            pl.BlockSpec((H, D), lambda b: (0, 0)),
            pl.BlockSpec((1, H), lambda b: (0, 0)),
        ],
        out_specs=pl.BlockSpec((1, N, H), lambda b: (b, 0, 0)),
        out_shape=jax.ShapeDtypeStruct((B, N, H), jnp.float32),
    )(region_features, W_node, b_node.reshape(1, H))

    import functools
    out, vals = pl.pallas_call(
        functools.partial(_main_kernel, R=R, K=K, N=N),
        grid=(B, N // R),
        in_specs=[
            pl.BlockSpec((1, N, H), lambda b, rb: (b, 0, 0)),
            pl.BlockSpec((H, H), lambda b, rb: (0, 0)),
            pl.BlockSpec((1, H), lambda b, rb: (0, 0)),
            pl.BlockSpec((D, H), lambda b, rb: (0, 0)),
            pl.BlockSpec((1, D), lambda b, rb: (0, 0)),
        ],
        out_specs=[
            pl.BlockSpec((1, R, D), lambda b, rb: (b, rb, 0)),
            pl.BlockSpec((1, R, K), lambda b, rb: (b, rb, 0)),
        ],
        out_shape=[
            jax.ShapeDtypeStruct((B, N, D), jnp.float32),
            jax.ShapeDtypeStruct((B, N, K), jnp.float32),
        ],
    )(nodes, W_msg, b_msg.reshape(1, H), W_out, b_out.reshape(1, D))

    return (out, vals)


# speedup vs baseline: 2.1632x; 2.1632x over previous
import functools
import jax
import jax.numpy as jnp
from jax.experimental import pallas as pl
from jax.experimental.pallas import tpu as pltpu


def _fused_kernel(rf_ref, wnode_ref, bnode_ref, wmsg_ref, bmsg_ref,
                  wout_ref, bout_ref, out_ref, vals_ref, nodes_vmem,
                  *, R, K, N):
    rb = pl.program_id(1)

    @pl.when(rb == 0)
    def _():
        rf = rf_ref[0]
        nodes_vmem[...] = jax.lax.dot_general(
            rf, wnode_ref[...], (((1,), (1,)), ((), ())),
            preferred_element_type=jnp.float32) + bnode_ref[...]

    nodes_all = nodes_vmem[...]
    rows = nodes_vmem[pl.ds(rb * R, R), :]

    sim = jax.lax.dot_general(
        rows, nodes_all, (((1,), (1,)), ((), ())),
        preferred_element_type=jnp.float32)

    col = jax.lax.broadcasted_iota(jnp.int32, (R, N), 1)
    row_g = rb * R + jax.lax.broadcasted_iota(jnp.int32, (R, N), 0)
    sim = jnp.where(col == row_g, jnp.float32(-1e9), sim)

    lane_k = jax.lax.broadcasted_iota(jnp.int32, (R, K), 1)
    vals = jnp.zeros((R, K), jnp.float32)
    m = jnp.float32(0)
    for kk in range(K):
        work = sim if kk == 0 else jnp.where(sim < m, sim, -jnp.inf)
        m = jnp.max(work, axis=1, keepdims=True)
        vals = jnp.where(lane_k == kk, m, vals)
    sel = (sim >= m).astype(jnp.float32)

    neigh = jax.lax.dot_general(
        sel, nodes_all, (((1,), (0,)), ((), ())),
        preferred_element_type=jnp.float32) * jnp.float32(1.0 / K)

    msgs = jax.lax.dot_general(
        neigh, wmsg_ref[...], (((1,), (1,)), ((), ())),
        preferred_element_type=jnp.float32) + bmsg_ref[...]
    msgs = jnp.maximum(msgs, 0.0)
    updated = rows + msgs

    out = jax.lax.dot_general(
        updated, wout_ref[...], (((1,), (1,)), ((), ())),
        preferred_element_type=jnp.float32) + bout_ref[...]
    out_ref[0] = out
    vals_ref[0] = vals


def kernel(region_features, W_node, b_node, W_msg, b_msg, W_out, b_out):
    B, N, D = region_features.shape
    H = W_node.shape[0]
    K = min(6, N - 1)
    R = 128

    out, vals = pl.pallas_call(
        functools.partial(_fused_kernel, R=R, K=K, N=N),
        grid=(B, N // R),
        in_specs=[
            pl.BlockSpec((1, N, D), lambda b, rb: (b, 0, 0)),
            pl.BlockSpec((H, D), lambda b, rb: (0, 0)),
            pl.BlockSpec((1, H), lambda b, rb: (0, 0)),
            pl.BlockSpec((H, H), lambda b, rb: (0, 0)),
            pl.BlockSpec((1, H), lambda b, rb: (0, 0)),
            pl.BlockSpec((D, H), lambda b, rb: (0, 0)),
            pl.BlockSpec((1, D), lambda b, rb: (0, 0)),
        ],
        out_specs=[
            pl.BlockSpec((1, R, D), lambda b, rb: (b, rb, 0)),
            pl.BlockSpec((1, R, K), lambda b, rb: (b, rb, 0)),
        ],
        out_shape=[
            jax.ShapeDtypeStruct((B, N, D), jnp.float32),
            jax.ShapeDtypeStruct((B, N, K), jnp.float32),
        ],
        scratch_shapes=[pltpu.VMEM((N, H), jnp.float32)],
        compiler_params=pltpu.CompilerParams(
            dimension_semantics=("arbitrary", "arbitrary")),
    )(region_features, W_node, b_node.reshape(1, H), W_msg,
      b_msg.reshape(1, H), W_out, b_out.reshape(1, D))

    return (out, vals)


# speedup vs baseline: 2.2271x; 1.0295x over previous
import functools
import jax
import jax.numpy as jnp
from jax.experimental import pallas as pl
from jax.experimental.pallas import tpu as pltpu


def _subtile(nodes_vmem, wmsg_ref, bmsg_ref, wout_ref, bout_ref, rbt, R, K, N):
    nodes_all = nodes_vmem[...]
    rows = nodes_vmem[pl.ds(rbt * R, R), :]

    sim = jax.lax.dot_general(
        rows, nodes_all, (((1,), (1,)), ((), ())),
        preferred_element_type=jnp.float32)

    col = jax.lax.broadcasted_iota(jnp.int32, (R, N), 1)
    row_g = rbt * R + jax.lax.broadcasted_iota(jnp.int32, (R, N), 0)
    sim = jnp.where(col == row_g, jnp.float32(-1e9), sim)

    lane_k = jax.lax.broadcasted_iota(jnp.int32, (R, K), 1)
    vals = jnp.zeros((R, K), jnp.float32)
    m = jnp.float32(0)
    for kk in range(K):
        work = sim if kk == 0 else jnp.where(sim < m, sim, -jnp.inf)
        m = jnp.max(work, axis=1, keepdims=True)
        vals = jnp.where(lane_k == kk, m, vals)
    sel = (sim >= m).astype(jnp.float32)

    neigh = jax.lax.dot_general(
        sel, nodes_all, (((1,), (0,)), ((), ())),
        preferred_element_type=jnp.float32) * jnp.float32(1.0 / K)

    msgs = jax.lax.dot_general(
        neigh, wmsg_ref[...], (((1,), (1,)), ((), ())),
        preferred_element_type=jnp.float32) + bmsg_ref[...]
    msgs = jnp.maximum(msgs, 0.0)
    updated = rows + msgs

    out = jax.lax.dot_general(
        updated, wout_ref[...], (((1,), (1,)), ((), ())),
        preferred_element_type=jnp.float32) + bout_ref[...]
    return out, vals


def _fused_kernel(rf_ref, wnode_ref, bnode_ref, wmsg_ref, bmsg_ref,
                  wout_ref, bout_ref, out_ref, vals_ref, nodes_vmem,
                  *, R, K, N, T):
    rb = pl.program_id(1)

    @pl.when(rb == 0)
    def _():
        rf = rf_ref[0]
        nodes_vmem[...] = jax.lax.dot_general(
            rf, wnode_ref[...], (((1,), (1,)), ((), ())),
            preferred_element_type=jnp.float32) + bnode_ref[...]

    for t in range(T):
        out_t, vals_t = _subtile(nodes_vmem, wmsg_ref, bmsg_ref,
                                 wout_ref, bout_ref, rb * T + t, R, K, N)
        out_ref[0, t * R:(t + 1) * R, :] = out_t
        vals_ref[0, t * R:(t + 1) * R, :] = vals_t


def kernel(region_features, W_node, b_node, W_msg, b_msg, W_out, b_out):
    B, N, D = region_features.shape
    H = W_node.shape[0]
    K = min(6, N - 1)
    R = 128
    T = 2

    out, vals = pl.pallas_call(
        functools.partial(_fused_kernel, R=R, K=K, N=N, T=T),
        grid=(B, N // (R * T)),
        in_specs=[
            pl.BlockSpec((1, N, D), lambda b, rb: (b, 0, 0)),
            pl.BlockSpec((H, D), lambda b, rb: (0, 0)),
            pl.BlockSpec((1, H), lambda b, rb: (0, 0)),
            pl.BlockSpec((H, H), lambda b, rb: (0, 0)),
            pl.BlockSpec((1, H), lambda b, rb: (0, 0)),
            pl.BlockSpec((D, H), lambda b, rb: (0, 0)),
            pl.BlockSpec((1, D), lambda b, rb: (0, 0)),
        ],
        out_specs=[
            pl.BlockSpec((1, R * T, D), lambda b, rb: (b, rb, 0)),
            pl.BlockSpec((1, R * T, K), lambda b, rb: (b, rb, 0)),
        ],
        out_shape=[
            jax.ShapeDtypeStruct((B, N, D), jnp.float32),
            jax.ShapeDtypeStruct((B, N, K), jnp.float32),
        ],
        scratch_shapes=[pltpu.VMEM((N, H), jnp.float32)],
        compiler_params=pltpu.CompilerParams(
            dimension_semantics=("arbitrary", "arbitrary")),
    )(region_features, W_node, b_node.reshape(1, H), W_msg,
      b_msg.reshape(1, H), W_out, b_out.reshape(1, D))

    return (out, vals)
